# baseline (speedup 1.0000x reference)
import jax
import jax.numpy as jnp
from jax.experimental import pallas as pl
from jax.experimental.pallas import tpu as pltpu

_NUM_CLASSES = 1000
_BATCH = 16384
_BLOCK_ROWS = 512
_NBUF = 4
_NSTEPS = _BATCH // _BLOCK_ROWS


def _onehot_body(x_ref, o_ref, buf, sems):
    i = pl.program_id(0)
    slot = jax.lax.rem(i, _NBUF)

    @pl.when(i >= _NBUF)
    def _wait_prev():
        pltpu.make_async_copy(
            buf.at[slot],
            o_ref.at[pl.ds((i - _NBUF) * _BLOCK_ROWS, _BLOCK_ROWS)],
            sems.at[slot],
        ).wait()

    ids = x_ref[...]
    cols = jax.lax.broadcasted_iota(
        jnp.int32, (_BLOCK_ROWS, _NUM_CLASSES), 1
    )
    buf[slot] = (cols == ids).astype(jnp.float32)
    pltpu.make_async_copy(
        buf.at[slot],
        o_ref.at[pl.ds(i * _BLOCK_ROWS, _BLOCK_ROWS)],
        sems.at[slot],
    ).start()

    @pl.when(i == _NSTEPS - 1)
    def _drain():
        for s in range(_NBUF):
            pltpu.make_async_copy(
                buf.at[s],
                o_ref.at[pl.ds(0, _BLOCK_ROWS)],
                sems.at[s],
            ).wait()


def kernel(x1):
    x = x1.astype(jnp.int32).reshape(_BATCH, 1)
    return pl.pallas_call(
        _onehot_body,
        grid=(_NSTEPS,),
        in_specs=[pl.BlockSpec((_BLOCK_ROWS, 1), lambda i: (i, 0))],
        out_specs=pl.BlockSpec(memory_space=pltpu.MemorySpace.HBM),
        out_shape=jax.ShapeDtypeStruct((_BATCH, _NUM_CLASSES), jnp.float32),
        scratch_shapes=[
            pltpu.VMEM((_NBUF, _BLOCK_ROWS, _NUM_CLASSES), jnp.float32),
            pltpu.SemaphoreType.DMA((_NBUF,)),
        ],
    )(x)
